# baseline (speedup 1.0000x reference)
import dataclasses
import functools

import jax
import jax.numpy as jnp
from jax import lax
from jax.experimental import pallas as pl
from jax.experimental.pallas import tpu as pltpu
from jax.experimental.pallas import tpu_sc as plsc

_N_NODES = 10000
_N_EDGES = 320000
_D = 128
_NC = 2
_NS = 16
_NW = _NC * _NS
_B = 48
_NBLK = 212
_EPW = _B * _NBLK
_E_PAD = _EPW * _NW
_ACC_ROWS = 10240
_ZROWS = _ACC_ROWS // _NS
_ZB = 32
_ZCH = _ZROWS // _ZB
_CNT = _ACC_ROWS // 2


def _matmul_kernel(x_ref, w_ref, o_ref):
    o_ref[...] = jnp.dot(x_ref[...], w_ref[...],
                         preferred_element_type=jnp.float32)


def _edge_mlp_kernel(e_ref, w_ref, b_ref, o_ref):
    o_ref[...] = jnp.dot(e_ref[...], w_ref[...],
                         preferred_element_type=jnp.float32) + b_ref[...]


def _count_reduce_kernel(cw_ref, o_ref):
    cw = cw_ref[...]
    low = jnp.sum(lax.bitwise_and(cw, 0xFFFF), axis=0)
    high = jnp.sum(lax.shift_right_logical(cw, 16), axis=0)
    o_ref[...] = jnp.stack([low, high], axis=1).astype(jnp.float32)


def _node_mlp_kernel(x_ref, s0_ref, s1_ref, cnt_ref, batch_ref, u_ref,
                     w2_ref, b2_ref, w3a_ref, w3b_ref, w3c_ref, b3_ref,
                     w4_ref, b4_ref, o_ref):
    s = s0_ref[...] + s1_ref[...]
    cnt = cnt_ref[...]
    meanh = s / jnp.maximum(cnt, 1.0)
    m = (jnp.dot(meanh, w2_ref[...], preferred_element_type=jnp.float32)
         + jnp.where(cnt > 0.0, 1.0, 0.0) * b2_ref[...])
    onehot = (batch_ref[...] == lax.broadcasted_iota(
        jnp.int32, (batch_ref.shape[0], 8), 1)).astype(jnp.float32)
    ug = jnp.dot(onehot,
                 jnp.dot(u_ref[...], w3c_ref[...],
                         preferred_element_type=jnp.float32),
                 preferred_element_type=jnp.float32)
    h2 = jnp.maximum(
        jnp.dot(x_ref[...], w3a_ref[...], preferred_element_type=jnp.float32)
        + jnp.dot(m, w3b_ref[...], preferred_element_type=jnp.float32)
        + ug + b3_ref[...], 0.0)
    o_ref[...] = (jnp.dot(h2, w4_ref[...], preferred_element_type=jnp.float32)
                  + b4_ref[...])


def _sc_edge_kernel(xw_hbm, rc_hbm, ea_hbm, zs_hbm, rid_hbm,
                    outs_hbm, outc_hbm,
                    ridx, rc0, rc1, rc2, rc3, gbufa, gbufb, eabufa, eabufb,
                    cnt, acc_s,
                    sga, sgb, sea, seb, ssa, ssb, si0, si1, si2, si3):
    cid = lax.axis_index("c")
    sid = lax.axis_index("s")
    wid = cid * _NS + sid

    pltpu.sync_copy(zs_hbm, gbufa)

    @pl.loop(0, _ZCH)
    def _zero(k):
        pltpu.sync_copy(rid_hbm.at[pl.ds(sid * _ZROWS + k * _ZB, _ZB)], ridx)
        pltpu.sync_copy(gbufa.at[pl.ds(0, _ZB)], acc_s.at[ridx])

    zvec = jnp.zeros((16,), jnp.int32)

    @pl.loop(0, _CNT, step=16)
    def _zcnt(k):
        cnt[pl.ds(k, 16)] = zvec

    plsc.subcore_barrier()

    blk0 = wid * _NBLK
    rcs = (rc0, rc1, rc2, rc3)
    sis = (si0, si1, si2, si3)
    gbufs = (gbufa, gbufb)
    eabufs = (eabufa, eabufb)
    sgs = (sga, sgb)
    ses = (sea, seb)
    sss = (ssa, ssb)

    def idx_start(b, slot):
        pltpu.async_copy(rc_hbm.at[blk0 + b], rcs[slot], sis[slot])

    def idx_wait(b, slot):
        pltpu.make_async_copy(rc_hbm.at[blk0 + b], rcs[slot],
                              sis[slot]).wait()

    def gather_start(b, slot, p):
        pltpu.async_copy(xw_hbm.at[rcs[slot].at[0]], gbufs[p], sgs[p])

    def gather_wait(b, slot, p):
        pltpu.make_async_copy(xw_hbm.at[rcs[slot].at[0]], gbufs[p],
                              sgs[p]).wait()

    def ea_start(b, p):
        pltpu.async_copy(ea_hbm.at[pl.ds((blk0 + b) * _B, _B)],
                         eabufs[p], ses[p])

    def ea_wait(b, p):
        pltpu.make_async_copy(ea_hbm.at[pl.ds((blk0 + b) * _B, _B)],
                              eabufs[p], ses[p]).wait()

    def scat_start(b, slot, p):
        pltpu.async_copy(eabufs[p], acc_s.at[rcs[slot].at[1]], sss[p],
                         add=True)

    def scat_wait(b, slot, p):
        pltpu.make_async_copy(eabufs[p], acc_s.at[rcs[slot].at[1]],
                              sss[p]).wait()

    pltpu.sync_copy(rc_hbm.at[blk0], rc0)
    idx_start(1, 1)
    gather_start(0, 0, 0)
    ea_start(0, 0)

    @pl.loop(0, _NBLK, step=4)
    def _blk(i):
        for k in range(4):
            b = i + k
            p = k % 2
            q = 1 - p
            slot = k
            nslot = (k + 1) % 4
            pslot = (k + 2) % 4

            @pl.when(b + 1 < _NBLK)
            def _():
                idx_wait(b + 1, nslot)
                gather_start(b + 1, nslot, q)

            @pl.when(b >= 1)
            def _():
                scat_wait(b - 1, (k + 3) % 4, q)

            @pl.when(b + 1 < _NBLK)
            def _():
                ea_start(b + 1, q)

            @pl.when(b + 2 < _NBLK)
            def _():
                idx_start(b + 2, pslot)

            gather_wait(b, slot, p)
            ea_wait(b, p)

            gbuf = gbufs[p]
            eabuf = eabufs[p]

            @pl.loop(0, _B)
            def _row(r):
                for c in range(8):
                    sl = pl.ds(c * 16, 16)
                    eabuf.at[r, sl][...] = jnp.maximum(
                        gbuf.at[r, sl][...] + eabuf.at[r, sl][...], 0.0)

            for k2 in range(_B // 16):
                cidx_v = rcs[slot].at[1, pl.ds(k2 * 16, 16)][...]
                half = lax.shift_right_logical(cidx_v, 1)
                addv = jnp.where(lax.bitwise_and(cidx_v, 1) == 1,
                                 jnp.int32(1 << 16), jnp.int32(1))
                plsc.addupdate_scatter(cnt, [half], addv)

            scat_start(b, slot, p)

    scat_wait(_NBLK - 1, (_NBLK - 1) % 4, (_NBLK - 1) % 2)
    plsc.subcore_barrier()

    @pl.loop(0, _ZCH)
    def _export(k):
        r = sid * _ZROWS + k * _ZB
        pltpu.sync_copy(rid_hbm.at[pl.ds(r, _ZB)], ridx)
        pltpu.sync_copy(acc_s.at[ridx], gbufa.at[pl.ds(0, _ZB)])
        pltpu.sync_copy(gbufa.at[pl.ds(0, _ZB)],
                        outs_hbm.at[pl.ds(cid * _ACC_ROWS + r, _ZB)])

    pltpu.sync_copy(cnt, outc_hbm.at[wid])


def _sc_edge_aggregate(xw, row, col, ea, zs):
    mesh = plsc.VectorSubcoreMesh(core_axis_name="c", subcore_axis_name="s")
    cp = pltpu.CompilerParams()
    if "needs_layout_passes" in pltpu.CompilerParams.__dataclass_fields__:
        cp = dataclasses.replace(cp, needs_layout_passes=False)
    f = pl.kernel(
        _sc_edge_kernel,
        compiler_params=cp,
        out_type=(jax.ShapeDtypeStruct((_NC * _ACC_ROWS, _D), jnp.float32),
                  jax.ShapeDtypeStruct((_NW, _CNT), jnp.int32)),
        mesh=mesh,
        scratch_types=[
            pltpu.VMEM((_ZB,), jnp.int32),
            pltpu.VMEM((2, _B), jnp.int32),
            pltpu.VMEM((2, _B), jnp.int32),
            pltpu.VMEM((2, _B), jnp.int32),
            pltpu.VMEM((2, _B), jnp.int32),
            pltpu.VMEM((_B, _D), jnp.float32),
            pltpu.VMEM((_B, _D), jnp.float32),
            pltpu.VMEM((_B, _D), jnp.float32),
            pltpu.VMEM((_B, _D), jnp.float32),
            pltpu.VMEM((_CNT,), jnp.int32),
            pltpu.VMEM_SHARED((_ACC_ROWS, _D), jnp.float32),
            pltpu.SemaphoreType.DMA,
            pltpu.SemaphoreType.DMA,
            pltpu.SemaphoreType.DMA,
            pltpu.SemaphoreType.DMA,
            pltpu.SemaphoreType.DMA,
            pltpu.SemaphoreType.DMA,
            pltpu.SemaphoreType.DMA,
            pltpu.SemaphoreType.DMA,
            pltpu.SemaphoreType.DMA,
            pltpu.SemaphoreType.DMA,
        ],
    )
    rc = jnp.stack([row, col], axis=0).reshape(
        2, _NW * _NBLK, _B).transpose(1, 0, 2)
    rid = jnp.arange(_ACC_ROWS, dtype=jnp.int32)
    return f(xw, rc, ea, zs, rid)


def kernel(x, edge_index, edge_attr, u, batch, W1, b1, W2, b2, W3, b3, W4, b4):
    npad = _E_PAD - _N_EDGES
    row = jnp.concatenate([edge_index[0].astype(jnp.int32),
                           jnp.zeros((npad,), jnp.int32)])
    col = jnp.concatenate([edge_index[1].astype(jnp.int32),
                           _N_NODES + (jnp.arange(npad, dtype=jnp.int32)
                                       % (_ACC_ROWS - _N_NODES))])
    eattr = jnp.concatenate([edge_attr,
                             jnp.zeros((npad, edge_attr.shape[1]),
                                       jnp.float32)])
    W1a = W1[:_D]
    W1b = W1[_D:]
    W3a = W3[:_D]
    W3b = W3[_D:2 * _D]
    W3c = W3[2 * _D:]

    xw = pl.pallas_call(
        _matmul_kernel,
        out_shape=jax.ShapeDtypeStruct((_N_NODES, _D), jnp.float32),
        grid=(5,),
        in_specs=[pl.BlockSpec((2000, _D), lambda i: (i, 0)),
                  pl.BlockSpec((_D, _D), lambda i: (0, 0))],
        out_specs=pl.BlockSpec((2000, _D), lambda i: (i, 0)),
    )(x, W1a)

    ea = pl.pallas_call(
        _edge_mlp_kernel,
        out_shape=jax.ShapeDtypeStruct((_E_PAD, _D), jnp.float32),
        grid=(_E_PAD // 2048,),
        in_specs=[pl.BlockSpec((2048, 16), lambda i: (i, 0)),
                  pl.BlockSpec((16, _D), lambda i: (0, 0)),
                  pl.BlockSpec((1, _D), lambda i: (0, 0))],
        out_specs=pl.BlockSpec((2048, _D), lambda i: (i, 0)),
    )(eattr, W1b, b1.reshape(1, _D))

    zs = jnp.zeros((_B, _D), jnp.float32)

    s2, cw = _sc_edge_aggregate(xw, row, col, ea, zs)
    s0 = s2[:_N_NODES]
    s1 = s2[_ACC_ROWS:_ACC_ROWS + _N_NODES]

    cnt = pl.pallas_call(
        _count_reduce_kernel,
        out_shape=jax.ShapeDtypeStruct((_CNT, 2), jnp.float32),
        grid=(_CNT // 1024,),
        in_specs=[pl.BlockSpec((_NW, 1024), lambda i: (0, i))],
        out_specs=pl.BlockSpec((1024, 2), lambda i: (i, 0)),
    )(cw).reshape(_ACC_ROWS, 1)[:_N_NODES]

    out = pl.pallas_call(
        _node_mlp_kernel,
        out_shape=jax.ShapeDtypeStruct((_N_NODES, 128), jnp.float32),
        grid=(5,),
        in_specs=[
            pl.BlockSpec((2000, _D), lambda i: (i, 0)),
            pl.BlockSpec((2000, _D), lambda i: (i, 0)),
            pl.BlockSpec((2000, _D), lambda i: (i, 0)),
            pl.BlockSpec((2000, 1), lambda i: (i, 0)),
            pl.BlockSpec((2000, 1), lambda i: (i, 0)),
            pl.BlockSpec((8, 16), lambda i: (0, 0)),
            pl.BlockSpec((_D, _D), lambda i: (0, 0)),
            pl.BlockSpec((1, _D), lambda i: (0, 0)),
            pl.BlockSpec((_D, _D), lambda i: (0, 0)),
            pl.BlockSpec((_D, _D), lambda i: (0, 0)),
            pl.BlockSpec((16, _D), lambda i: (0, 0)),
            pl.BlockSpec((1, _D), lambda i: (0, 0)),
            pl.BlockSpec((_D, 128), lambda i: (0, 0)),
            pl.BlockSpec((1, 128), lambda i: (0, 0)),
        ],
        out_specs=pl.BlockSpec((2000, 128), lambda i: (i, 0)),
    )(x, s0, s1, cnt, batch.astype(jnp.int32).reshape(_N_NODES, 1), u,
      W2, b2.reshape(1, _D), W3a, W3b, W3c, b3.reshape(1, _D),
      W4, b4.reshape(1, 128))
    return out


# speedup vs baseline: 1.1000x; 1.1000x over previous
import dataclasses
import functools

import jax
import jax.numpy as jnp
from jax import lax
from jax.experimental import pallas as pl
from jax.experimental.pallas import tpu as pltpu
from jax.experimental.pallas import tpu_sc as plsc

_N_NODES = 10000
_N_EDGES = 320000
_D = 128
_NC = 2
_NS = 16
_NW = _NC * _NS
_B = 32
_NBLK = 316
_EPW = _B * _NBLK
_E_PAD = _EPW * _NW
_ACC_ROWS = 10240
_ZROWS = _ACC_ROWS // _NS
_ZB = 32
_ZCH = _ZROWS // _ZB
_CNT = _ACC_ROWS // 2


def _matmul_kernel(x_ref, w_ref, o_ref):
    o_ref[...] = jnp.dot(x_ref[...], w_ref[...],
                         preferred_element_type=jnp.float32)


def _edge_mlp_kernel(e_ref, w_ref, b_ref, o_ref):
    o_ref[...] = jnp.dot(e_ref[...], w_ref[...],
                         preferred_element_type=jnp.float32) + b_ref[...]


def _count_reduce_kernel(cw_ref, o_ref):
    cw = cw_ref[...]
    low = jnp.sum(lax.bitwise_and(cw, 0xFFFF), axis=0)
    high = jnp.sum(lax.shift_right_logical(cw, 16), axis=0)
    o_ref[...] = jnp.stack([low, high], axis=1).astype(jnp.float32)


def _node_mlp_kernel(x_ref, s0_ref, s1_ref, cnt_ref, batch_ref, u_ref,
                     w2_ref, b2_ref, w3a_ref, w3b_ref, w3c_ref, b3_ref,
                     w4_ref, b4_ref, o_ref):
    s = s0_ref[...] + s1_ref[...]
    cnt = cnt_ref[...]
    meanh = s / jnp.maximum(cnt, 1.0)
    m = (jnp.dot(meanh, w2_ref[...], preferred_element_type=jnp.float32)
         + jnp.where(cnt > 0.0, 1.0, 0.0) * b2_ref[...])
    onehot = (batch_ref[...] == lax.broadcasted_iota(
        jnp.int32, (batch_ref.shape[0], 8), 1)).astype(jnp.float32)
    ug = jnp.dot(onehot,
                 jnp.dot(u_ref[...], w3c_ref[...],
                         preferred_element_type=jnp.float32),
                 preferred_element_type=jnp.float32)
    h2 = jnp.maximum(
        jnp.dot(x_ref[...], w3a_ref[...], preferred_element_type=jnp.float32)
        + jnp.dot(m, w3b_ref[...], preferred_element_type=jnp.float32)
        + ug + b3_ref[...], 0.0)
    o_ref[...] = (jnp.dot(h2, w4_ref[...], preferred_element_type=jnp.float32)
                  + b4_ref[...])


def _sc_edge_kernel(xw_hbm, rc_hbm, ea_hbm, zs_hbm, rid_hbm,
                    outs_hbm, outc_hbm,
                    ridx, rc0, rc1, rc2, rc3, gbufa, gbufb, eabufa, eabufb,
                    cnt, acc_s,
                    sga, sgb, sea, seb, ssa, ssb, si0, si1, si2, si3):
    cid = lax.axis_index("c")
    sid = lax.axis_index("s")
    wid = cid * _NS + sid

    pltpu.sync_copy(zs_hbm, gbufa)

    @pl.loop(0, _ZCH)
    def _zero(k):
        pltpu.sync_copy(rid_hbm.at[pl.ds(sid * _ZROWS + k * _ZB, _ZB)], ridx)
        pltpu.sync_copy(gbufa.at[pl.ds(0, _ZB)], acc_s.at[ridx])

    zvec = jnp.zeros((16,), jnp.int32)

    @pl.loop(0, _CNT, step=16)
    def _zcnt(k):
        cnt[pl.ds(k, 16)] = zvec

    plsc.subcore_barrier()

    blk0 = wid * _NBLK
    rcs = (rc0, rc1, rc2, rc3)
    sis = (si0, si1, si2, si3)
    gbufs = (gbufa, gbufb)
    eabufs = (eabufa, eabufb)
    sgs = (sga, sgb)
    ses = (sea, seb)
    sss = (ssa, ssb)

    def idx_start(b, slot):
        pltpu.async_copy(rc_hbm.at[blk0 + b], rcs[slot], sis[slot])

    def idx_wait(b, slot):
        pltpu.make_async_copy(rc_hbm.at[blk0 + b], rcs[slot],
                              sis[slot]).wait()

    def gather_start(b, slot, p):
        pltpu.async_copy(xw_hbm.at[rcs[slot].at[0]], gbufs[p], sgs[p])

    def gather_wait(b, slot, p):
        pltpu.make_async_copy(xw_hbm.at[rcs[slot].at[0]], gbufs[p],
                              sgs[p]).wait()

    def ea_start(b, p):
        pltpu.async_copy(ea_hbm.at[pl.ds((blk0 + b) * _B, _B)],
                         eabufs[p], ses[p])

    def ea_wait(b, p):
        pltpu.make_async_copy(ea_hbm.at[pl.ds((blk0 + b) * _B, _B)],
                              eabufs[p], ses[p]).wait()

    def scat_start(b, slot, p):
        pltpu.async_copy(eabufs[p], acc_s.at[rcs[slot].at[1]], sss[p],
                         add=True)

    def scat_wait(b, slot, p):
        pltpu.make_async_copy(eabufs[p], acc_s.at[rcs[slot].at[1]],
                              sss[p]).wait()

    pltpu.sync_copy(rc_hbm.at[blk0], rc0)
    idx_start(1, 1)
    gather_start(0, 0, 0)
    ea_start(0, 0)

    @pl.loop(0, _NBLK, step=4)
    def _blk(i):
        for k in range(4):
            b = i + k
            p = k % 2
            q = 1 - p
            slot = k
            nslot = (k + 1) % 4
            pslot = (k + 2) % 4

            @pl.when(b + 1 < _NBLK)
            def _():
                idx_wait(b + 1, nslot)
                gather_start(b + 1, nslot, q)

            @pl.when(b >= 1)
            def _():
                scat_wait(b - 1, (k + 3) % 4, q)

            @pl.when(b + 1 < _NBLK)
            def _():
                ea_start(b + 1, q)

            @pl.when(b + 2 < _NBLK)
            def _():
                idx_start(b + 2, pslot)

            gather_wait(b, slot, p)
            ea_wait(b, p)

            gbuf = gbufs[p]
            eabuf = eabufs[p]

            @pl.loop(0, _B)
            def _row(r):
                for c in range(8):
                    sl = pl.ds(c * 16, 16)
                    eabuf.at[r, sl][...] = jnp.maximum(
                        gbuf.at[r, sl][...] + eabuf.at[r, sl][...], 0.0)

            for k2 in range(_B // 16):
                cidx_v = rcs[slot].at[1, pl.ds(k2 * 16, 16)][...]
                half = lax.shift_right_logical(cidx_v, 1)
                addv = jnp.where(lax.bitwise_and(cidx_v, 1) == 1,
                                 jnp.int32(1 << 16), jnp.int32(1))
                plsc.addupdate_scatter(cnt, [half], addv)

            scat_start(b, slot, p)

    scat_wait(_NBLK - 1, (_NBLK - 1) % 4, (_NBLK - 1) % 2)
    plsc.subcore_barrier()

    @pl.loop(0, _ZCH)
    def _export(k):
        r = sid * _ZROWS + k * _ZB
        pltpu.sync_copy(rid_hbm.at[pl.ds(r, _ZB)], ridx)
        pltpu.sync_copy(acc_s.at[ridx], gbufa.at[pl.ds(0, _ZB)])
        pltpu.sync_copy(gbufa.at[pl.ds(0, _ZB)],
                        outs_hbm.at[pl.ds(cid * _ACC_ROWS + r, _ZB)])

    pltpu.sync_copy(cnt, outc_hbm.at[wid])


def _sc_edge_aggregate(xw, row, col, ea, zs):
    mesh = plsc.VectorSubcoreMesh(core_axis_name="c", subcore_axis_name="s")
    cp = pltpu.CompilerParams()
    if "needs_layout_passes" in pltpu.CompilerParams.__dataclass_fields__:
        cp = dataclasses.replace(cp, needs_layout_passes=False)
    f = pl.kernel(
        _sc_edge_kernel,
        compiler_params=cp,
        out_type=(jax.ShapeDtypeStruct((_NC * _ACC_ROWS, _D), jnp.float32),
                  jax.ShapeDtypeStruct((_NW, _CNT), jnp.int32)),
        mesh=mesh,
        scratch_types=[
            pltpu.VMEM((_ZB,), jnp.int32),
            pltpu.VMEM((2, _B), jnp.int32),
            pltpu.VMEM((2, _B), jnp.int32),
            pltpu.VMEM((2, _B), jnp.int32),
            pltpu.VMEM((2, _B), jnp.int32),
            pltpu.VMEM((_B, _D), jnp.float32),
            pltpu.VMEM((_B, _D), jnp.float32),
            pltpu.VMEM((_B, _D), jnp.float32),
            pltpu.VMEM((_B, _D), jnp.float32),
            pltpu.VMEM((_CNT,), jnp.int32),
            pltpu.VMEM_SHARED((_ACC_ROWS, _D), jnp.float32),
            pltpu.SemaphoreType.DMA,
            pltpu.SemaphoreType.DMA,
            pltpu.SemaphoreType.DMA,
            pltpu.SemaphoreType.DMA,
            pltpu.SemaphoreType.DMA,
            pltpu.SemaphoreType.DMA,
            pltpu.SemaphoreType.DMA,
            pltpu.SemaphoreType.DMA,
            pltpu.SemaphoreType.DMA,
            pltpu.SemaphoreType.DMA,
        ],
    )
    rc = jnp.stack([row, col], axis=0).reshape(
        2, _NW * _NBLK, _B).transpose(1, 0, 2)
    rid = jnp.arange(_ACC_ROWS, dtype=jnp.int32)
    return f(xw, rc, ea, zs, rid)


def kernel(x, edge_index, edge_attr, u, batch, W1, b1, W2, b2, W3, b3, W4, b4):
    npad = _E_PAD - _N_EDGES
    row = jnp.concatenate([edge_index[0].astype(jnp.int32),
                           jnp.zeros((npad,), jnp.int32)])
    col = jnp.concatenate([edge_index[1].astype(jnp.int32),
                           _N_NODES + (jnp.arange(npad, dtype=jnp.int32)
                                       % (_ACC_ROWS - _N_NODES))])
    eattr = jnp.concatenate([edge_attr,
                             jnp.zeros((npad, edge_attr.shape[1]),
                                       jnp.float32)])
    W1a = W1[:_D]
    W1b = W1[_D:]
    W3a = W3[:_D]
    W3b = W3[_D:2 * _D]
    W3c = W3[2 * _D:]

    xw = pl.pallas_call(
        _matmul_kernel,
        out_shape=jax.ShapeDtypeStruct((_N_NODES, _D), jnp.float32),
        grid=(5,),
        in_specs=[pl.BlockSpec((2000, _D), lambda i: (i, 0)),
                  pl.BlockSpec((_D, _D), lambda i: (0, 0))],
        out_specs=pl.BlockSpec((2000, _D), lambda i: (i, 0)),
    )(x, W1a)

    ea = pl.pallas_call(
        _edge_mlp_kernel,
        out_shape=jax.ShapeDtypeStruct((_E_PAD, _D), jnp.float32),
        grid=(_E_PAD // 2048,),
        in_specs=[pl.BlockSpec((2048, 16), lambda i: (i, 0)),
                  pl.BlockSpec((16, _D), lambda i: (0, 0)),
                  pl.BlockSpec((1, _D), lambda i: (0, 0))],
        out_specs=pl.BlockSpec((2048, _D), lambda i: (i, 0)),
    )(eattr, W1b, b1.reshape(1, _D))

    zs = jnp.zeros((_B, _D), jnp.float32)

    s2, cw = _sc_edge_aggregate(xw, row, col, ea, zs)
    s0 = s2[:_N_NODES]
    s1 = s2[_ACC_ROWS:_ACC_ROWS + _N_NODES]

    cnt = pl.pallas_call(
        _count_reduce_kernel,
        out_shape=jax.ShapeDtypeStruct((_CNT, 2), jnp.float32),
        grid=(_CNT // 1024,),
        in_specs=[pl.BlockSpec((_NW, 1024), lambda i: (0, i))],
        out_specs=pl.BlockSpec((1024, 2), lambda i: (i, 0)),
    )(cw).reshape(_ACC_ROWS, 1)[:_N_NODES]

    out = pl.pallas_call(
        _node_mlp_kernel,
        out_shape=jax.ShapeDtypeStruct((_N_NODES, 128), jnp.float32),
        grid=(5,),
        in_specs=[
            pl.BlockSpec((2000, _D), lambda i: (i, 0)),
            pl.BlockSpec((2000, _D), lambda i: (i, 0)),
            pl.BlockSpec((2000, _D), lambda i: (i, 0)),
            pl.BlockSpec((2000, 1), lambda i: (i, 0)),
            pl.BlockSpec((2000, 1), lambda i: (i, 0)),
            pl.BlockSpec((8, 16), lambda i: (0, 0)),
            pl.BlockSpec((_D, _D), lambda i: (0, 0)),
            pl.BlockSpec((1, _D), lambda i: (0, 0)),
            pl.BlockSpec((_D, _D), lambda i: (0, 0)),
            pl.BlockSpec((_D, _D), lambda i: (0, 0)),
            pl.BlockSpec((16, _D), lambda i: (0, 0)),
            pl.BlockSpec((1, _D), lambda i: (0, 0)),
            pl.BlockSpec((_D, 128), lambda i: (0, 0)),
            pl.BlockSpec((1, 128), lambda i: (0, 0)),
        ],
        out_specs=pl.BlockSpec((2000, 128), lambda i: (i, 0)),
    )(x, s0, s1, cnt, batch.astype(jnp.int32).reshape(_N_NODES, 1), u,
      W2, b2.reshape(1, _D), W3a, W3b, W3c, b3.reshape(1, _D),
      W4, b4.reshape(1, 128))
    return out
